# baseline (speedup 1.0000x reference)
import dataclasses
import functools

import jax
import jax.numpy as jnp
from jax import lax
from jax.experimental import pallas as pl
from jax.experimental.pallas import tpu as pltpu
from jax.experimental.pallas import tpu_sc as plsc

_P = 3
_Q = 3
_G = 256
_M = 64
_B = 32
_L = 16
_CHUNK = 16
_NCHUNK = _G // _CHUNK


def _splat(val, dtype=jnp.int32):
    return jnp.full((_L,), val, dtype)


def _sc_body(ctrl_hbm, nut_hbm, nvt_hbm, uspan_hbm, vspan_hbm, out_hbm,
             ctrl_v, nut_v, nvt_v, uspan_v, vspan_v, tmp_v, obuf_v, sem):
    c = lax.axis_index("c")
    s = lax.axis_index("s")
    b = s * 2 + c

    pltpu.sync_copy(ctrl_hbm.at[b], ctrl_v)
    pltpu.sync_copy(nut_hbm, nut_v)
    pltpu.sync_copy(nvt_hbm, nvt_v)
    pltpu.sync_copy(uspan_hbm, uspan_v)
    pltpu.sync_copy(vspan_hbm, vspan_v)

    lane = jax.lax.iota(jnp.int32, _L)

    def _drain_one():
        pltpu.make_async_copy(
            obuf_v.at[pl.ds(0, _CHUNK)],
            out_hbm.at[b, pl.ds(0, _CHUNK)], sem).wait()

    @pl.loop(0, _NCHUNK)
    def _chunk(ic):
        @pl.when(ic >= 2)
        def _():
            _drain_one()
        @pl.loop(0, _CHUNK)
        def _rowa(ii):
            i = ic * _CHUNK + ii
            i_spl = _splat(0) + i
            ii_spl = _splat(0) + ii
            u0 = plsc.load_gather(uspan_v, [i_spl]) - _P
            nur = [plsc.load_gather(nut_v, [_splat(r), i_spl])
                   for r in range(_P + 1)]
            for ng in range(_M // _L):
                nidx = lane + ng * _L
                for d in range(4):
                    d_spl = _splat(d)
                    acc = nur[0] * plsc.load_gather(ctrl_v, [d_spl, u0, nidx])
                    for r in range(1, _P + 1):
                        acc = acc + nur[r] * plsc.load_gather(
                            ctrl_v, [d_spl, u0 + r, nidx])
                    plsc.store_scatter(tmp_v, [d_spl, ii_spl, nidx], acc)

        @pl.loop(0, _G // _L)
        def _grp(g):
            vs = plsc.load_gather(vspan_v, [lane + g * _L]) - _Q
            idx_s = [vs + s_ for s_ in range(_Q + 1)]
            nvs = [plsc.load_gather(nvt_v, [_splat(s_), lane + g * _L])
                   for s_ in range(_Q + 1)]
            lane3 = lane * 3 + g * (3 * _L)

            @pl.loop(0, _CHUNK)
            def _rowb(ii):
                ii_spl = _splat(0) + ii
                accs = []
                for d in range(4):
                    d_spl = _splat(d)
                    acc = nvs[0] * plsc.load_gather(
                        tmp_v, [d_spl, ii_spl, idx_s[0]])
                    for s_ in range(1, _Q + 1):
                        acc = acc + nvs[s_] * plsc.load_gather(
                            tmp_v, [d_spl, ii_spl, idx_s[s_]])
                    accs.append(acc)
                rw = 1.0 / accs[3]
                ib_spl = ii_spl + (ic % 2) * _CHUNK
                for d in range(3):
                    plsc.store_scatter(obuf_v, [ib_spl, lane3 + d],
                                       accs[d] * rw)

        pltpu.async_copy(
            obuf_v.at[pl.ds((ic % 2) * _CHUNK, _CHUNK)],
            out_hbm.at[b, pl.ds(ic * _CHUNK, _CHUNK)], sem)

    _drain_one()
    _drain_one()


def kernel(ctrl_pts, Nu_uv, Nv_uv, uspan_uv, vspan_uv):
    ctrl_t = ctrl_pts.transpose(0, 3, 1, 2)
    nut = Nu_uv.T
    nvt = Nv_uv.T

    mesh = plsc.VectorSubcoreMesh(core_axis_name="c", subcore_axis_name="s")
    cp = pltpu.CompilerParams()
    if "needs_layout_passes" in pltpu.CompilerParams.__dataclass_fields__:
        cp = dataclasses.replace(cp, needs_layout_passes=False)

    @functools.partial(
        pl.kernel,
        mesh=mesh,
        compiler_params=cp,
        out_type=jax.ShapeDtypeStruct((_B, _G, 3 * _G), jnp.float32),
        scratch_types=[
            pltpu.VMEM((4, _M, _M), jnp.float32),
            pltpu.VMEM((4, _G), jnp.float32),
            pltpu.VMEM((4, _G), jnp.float32),
            pltpu.VMEM((_G,), jnp.int32),
            pltpu.VMEM((_G,), jnp.int32),
            pltpu.VMEM((4, _CHUNK, _M), jnp.float32),
            pltpu.VMEM((2 * _CHUNK, 3 * _G), jnp.float32),
            pltpu.SemaphoreType.DMA,
        ],
    )
    def sc_eval(ctrl_hbm, nut_hbm, nvt_hbm, uspan_hbm, vspan_hbm, out_hbm,
                ctrl_v, nut_v, nvt_v, uspan_v, vspan_v, tmp_v, obuf_v, sem):
        _sc_body(ctrl_hbm, nut_hbm, nvt_hbm, uspan_hbm, vspan_hbm, out_hbm,
                 ctrl_v, nut_v, nvt_v, uspan_v, vspan_v, tmp_v, obuf_v, sem)

    out = sc_eval(ctrl_t, nut, nvt, uspan_uv, vspan_uv)
    return out.reshape(_B, _G, _G, 3)


# speedup vs baseline: 1.0940x; 1.0940x over previous
import dataclasses
import functools

import jax
import jax.numpy as jnp
from jax import lax
from jax.experimental import pallas as pl
from jax.experimental.pallas import tpu as pltpu
from jax.experimental.pallas import tpu_sc as plsc

_P = 3
_Q = 3
_G = 256
_M = 64
_B = 32
_L = 16
_CHUNK = 16
_NCHUNK = _G // _CHUNK


def _splat(val, dtype=jnp.int32):
    return jnp.full((_L,), val, dtype)


def _sc_body(ctrl_hbm, nut_hbm, nvt_hbm, uspan_hbm, vspan_hbm, out_hbm,
             ctrl_v, nut_v, nvt_v, uspan_v, vspan_v, tmp_v, obuf_v, sem):
    c = lax.axis_index("c")
    s = lax.axis_index("s")
    b = s * 2 + c

    pltpu.sync_copy(ctrl_hbm.at[b], ctrl_v)
    pltpu.sync_copy(nut_hbm, nut_v)
    pltpu.sync_copy(nvt_hbm, nvt_v)
    pltpu.sync_copy(uspan_hbm, uspan_v)
    pltpu.sync_copy(vspan_hbm, vspan_v)

    lane = jax.lax.iota(jnp.int32, _L)

    def _drain_one():
        pltpu.make_async_copy(
            obuf_v.at[pl.ds(0, _CHUNK)],
            out_hbm.at[b, pl.ds(0, _CHUNK)], sem).wait()

    @pl.loop(0, _NCHUNK)
    def _chunk(ic):
        @pl.when(ic >= 2)
        def _():
            _drain_one()
        @plsc.parallel_loop(0, _CHUNK, unroll=2)
        def _rowa(ii):
            i = ic * _CHUNK + ii
            i_spl = _splat(0) + i
            ii_spl = _splat(0) + ii
            u0 = plsc.load_gather(uspan_v, [i_spl]) - _P
            nur = [plsc.load_gather(nut_v, [_splat(r), i_spl])
                   for r in range(_P + 1)]
            for ng in range(_M // _L):
                nidx = lane + ng * _L
                for d in range(4):
                    d_spl = _splat(d)
                    acc = nur[0] * plsc.load_gather(ctrl_v, [d_spl, u0, nidx])
                    for r in range(1, _P + 1):
                        acc = acc + nur[r] * plsc.load_gather(
                            ctrl_v, [d_spl, u0 + r, nidx])
                    plsc.store_scatter(tmp_v, [d_spl, ii_spl, nidx], acc)

        @pl.loop(0, _G // _L)
        def _grp(g):
            vs = plsc.load_gather(vspan_v, [lane + g * _L]) - _Q
            idx_s = [vs + s_ for s_ in range(_Q + 1)]
            nvs = [plsc.load_gather(nvt_v, [_splat(s_), lane + g * _L])
                   for s_ in range(_Q + 1)]
            lane3 = lane * 3 + g * (3 * _L)

            @plsc.parallel_loop(0, _CHUNK, unroll=4)
            def _rowb(ii):
                ii_spl = _splat(0) + ii
                accs = []
                for d in range(4):
                    d_spl = _splat(d)
                    acc = nvs[0] * plsc.load_gather(
                        tmp_v, [d_spl, ii_spl, idx_s[0]])
                    for s_ in range(1, _Q + 1):
                        acc = acc + nvs[s_] * plsc.load_gather(
                            tmp_v, [d_spl, ii_spl, idx_s[s_]])
                    accs.append(acc)
                rw = 1.0 / accs[3]
                ib_spl = ii_spl + (ic % 2) * _CHUNK
                for d in range(3):
                    plsc.store_scatter(obuf_v, [ib_spl, lane3 + d],
                                       accs[d] * rw)

        pltpu.async_copy(
            obuf_v.at[pl.ds((ic % 2) * _CHUNK, _CHUNK)],
            out_hbm.at[b, pl.ds(ic * _CHUNK, _CHUNK)], sem)

    _drain_one()
    _drain_one()


def kernel(ctrl_pts, Nu_uv, Nv_uv, uspan_uv, vspan_uv):
    ctrl_t = ctrl_pts.transpose(0, 3, 1, 2)
    nut = Nu_uv.T
    nvt = Nv_uv.T

    mesh = plsc.VectorSubcoreMesh(core_axis_name="c", subcore_axis_name="s")
    cp = pltpu.CompilerParams()
    if "needs_layout_passes" in pltpu.CompilerParams.__dataclass_fields__:
        cp = dataclasses.replace(cp, needs_layout_passes=False)

    @functools.partial(
        pl.kernel,
        mesh=mesh,
        compiler_params=cp,
        out_type=jax.ShapeDtypeStruct((_B, _G, 3 * _G), jnp.float32),
        scratch_types=[
            pltpu.VMEM((4, _M, _M), jnp.float32),
            pltpu.VMEM((4, _G), jnp.float32),
            pltpu.VMEM((4, _G), jnp.float32),
            pltpu.VMEM((_G,), jnp.int32),
            pltpu.VMEM((_G,), jnp.int32),
            pltpu.VMEM((4, _CHUNK, _M), jnp.float32),
            pltpu.VMEM((2 * _CHUNK, 3 * _G), jnp.float32),
            pltpu.SemaphoreType.DMA,
        ],
    )
    def sc_eval(ctrl_hbm, nut_hbm, nvt_hbm, uspan_hbm, vspan_hbm, out_hbm,
                ctrl_v, nut_v, nvt_v, uspan_v, vspan_v, tmp_v, obuf_v, sem):
        _sc_body(ctrl_hbm, nut_hbm, nvt_hbm, uspan_hbm, vspan_hbm, out_hbm,
                 ctrl_v, nut_v, nvt_v, uspan_v, vspan_v, tmp_v, obuf_v, sem)

    out = sc_eval(ctrl_t, nut, nvt, uspan_uv, vspan_uv)
    return out.reshape(_B, _G, _G, 3)
